# baseline (speedup 1.0000x reference)
import functools

import jax
import jax.numpy as jnp
from jax import lax
from jax.experimental import pallas as pl
from jax.experimental.pallas import tpu as pltpu
from jax.experimental.pallas import tpu_sc as plsc

_LANES = 16
_NBUF = 6
_CHUNK = 16
_AHEAD = 3


@functools.lru_cache(maxsize=None)
def _build(B, S, D, V):
    info = plsc.get_sparse_core_info()
    NC, NS = info.num_cores, info.num_subcores
    NW = NC * NS
    assert S % (NW * _CHUNK) == 0 and D % _LANES == 0
    s_per_w = S // NW
    n_slices = D // _LANES
    n_chunks = B * s_per_w // _CHUNK

    mesh = plsc.VectorSubcoreMesh(core_axis_name="c", subcore_axis_name="s")

    @functools.partial(
        pl.kernel,
        mesh=mesh,
        out_type=jax.ShapeDtypeStruct((B * S, D), jnp.float32),
        scratch_types=(
            [pltpu.VMEM((B, s_per_w), jnp.int32),
             pltpu.VMEM((s_per_w, D), jnp.float32)]
            + [pltpu.VMEM((_CHUNK, D), jnp.float32) for _ in range(_NBUF)]
            + [pltpu.SemaphoreType.DMA for _ in range(2 + 2 * _NBUF)]
        ),
    )
    def emb(x_hbm, tok_hbm, pos_hbm, out_hbm, idx_all, pos_v, *rest):
        toks = list(rest[:_NBUF])
        psem, xsem = rest[_NBUF], rest[_NBUF + 1]
        gsems = list(rest[_NBUF + 2:2 * _NBUF + 2])
        ssems = list(rest[2 * _NBUF + 2:])
        wid = lax.axis_index("s") * NC + lax.axis_index("c")
        sbase = wid * s_per_w
        pos_cp = pltpu.async_copy(pos_hbm.at[pl.ds(sbase, s_per_w)], pos_v, psem)
        idx_cps = [
            pltpu.async_copy(x_hbm.at[pl.ds(b * S + sbase, s_per_w)],
                             idx_all.at[b], xsem)
            for b in range(B)
        ]
        for cp in idx_cps:
            cp.wait()

        chunks_per_row = s_per_w // _CHUNK

        def gather(c):
            b, h = divmod(c, chunks_per_row)
            idx_ref = idx_all.at[b, pl.ds(h * _CHUNK, _CHUNK)]
            return pltpu.async_copy(tok_hbm.at[idx_ref], toks[c % _NBUF],
                                    gsems[c % _NBUF])

        gd = {g: gather(g) for g in range(_AHEAD)}
        sd = {}
        for c in range(n_chunks):
            g = c + _AHEAD
            if g < n_chunks:
                if g - _NBUF >= 0:
                    sd[g - _NBUF].wait()
                gd[g] = gather(g)
            if c == 0:
                pos_cp.wait()
            gd[c].wait()
            b, h = divmod(c, chunks_per_row)
            tbuf = toks[c % _NBUF]

            def row(r, _, h=h, tbuf=tbuf):
                for j in range(n_slices):
                    sl = pl.ds(j * _LANES, _LANES)
                    plsc.addupdate(tbuf.at[r, sl],
                                   pos_v[h * _CHUNK + r, sl])
                return 0

            lax.fori_loop(0, _CHUNK, row, 0)
            dst = out_hbm.at[pl.ds(b * S + sbase + h * _CHUNK, _CHUNK)]
            sd[c] = pltpu.async_copy(tbuf, dst, ssems[c % _NBUF])
        for c in range(n_chunks - _NBUF, n_chunks):
            sd[c].wait()

    return emb


def kernel(x, tok_table, pos_table):
    B, S = x.shape
    V, D = tok_table.shape
    out_flat = _build(B, S, D, V)(x.reshape(B * S), tok_table, pos_table)
    return out_flat.reshape(B, S, D)


# speedup vs baseline: 1.1463x; 1.1463x over previous
import functools

import jax
import jax.numpy as jnp
from jax import lax
from jax.experimental import pallas as pl
from jax.experimental.pallas import tpu as pltpu
from jax.experimental.pallas import tpu_sc as plsc

_LANES = 16
_NBUF = 3
_CHUNK = 32
_AHEAD = 1


@functools.lru_cache(maxsize=None)
def _build(B, S, D, V):
    info = plsc.get_sparse_core_info()
    NC, NS = info.num_cores, info.num_subcores
    NW = NC * NS
    assert S % (NW * _CHUNK) == 0 and D % _LANES == 0
    s_per_w = S // NW
    n_slices = D // _LANES
    n_chunks = B * s_per_w // _CHUNK

    mesh = plsc.VectorSubcoreMesh(core_axis_name="c", subcore_axis_name="s")

    @functools.partial(
        pl.kernel,
        mesh=mesh,
        out_type=jax.ShapeDtypeStruct((B * S, D), jnp.float32),
        scratch_types=(
            [pltpu.VMEM((B, s_per_w), jnp.int32),
             pltpu.VMEM((s_per_w, D), jnp.float32)]
            + [pltpu.VMEM((_CHUNK, D), jnp.float32) for _ in range(_NBUF)]
            + [pltpu.SemaphoreType.DMA for _ in range(2 + 2 * _NBUF)]
        ),
    )
    def emb(x_hbm, tok_hbm, pos_hbm, out_hbm, idx_all, pos_v, *rest):
        toks = list(rest[:_NBUF])
        psem, xsem = rest[_NBUF], rest[_NBUF + 1]
        gsems = list(rest[_NBUF + 2:2 * _NBUF + 2])
        ssems = list(rest[2 * _NBUF + 2:])
        wid = lax.axis_index("s") * NC + lax.axis_index("c")
        sbase = wid * s_per_w
        pos_cp = pltpu.async_copy(pos_hbm.at[pl.ds(sbase, s_per_w)], pos_v, psem)
        idx_cps = [
            pltpu.async_copy(x_hbm.at[pl.ds(b * S + sbase, s_per_w)],
                             idx_all.at[b], xsem)
            for b in range(B)
        ]
        for cp in idx_cps:
            cp.wait()

        chunks_per_row = s_per_w // _CHUNK

        def gather(c):
            b, h = divmod(c, chunks_per_row)
            idx_ref = idx_all.at[b, pl.ds(h * _CHUNK, _CHUNK)]
            return pltpu.async_copy(tok_hbm.at[idx_ref], toks[c % _NBUF],
                                    gsems[c % _NBUF])

        gd = {g: gather(g) for g in range(_AHEAD)}
        sd = {}
        for c in range(n_chunks):
            g = c + _AHEAD
            if g < n_chunks:
                if g - _NBUF >= 0:
                    sd[g - _NBUF].wait()
                gd[g] = gather(g)
            if c == 0:
                pos_cp.wait()
            gd[c].wait()
            b, h = divmod(c, chunks_per_row)
            tbuf = toks[c % _NBUF]

            @plsc.parallel_loop(0, _CHUNK, step=1, unroll=2)
            def row(r, h=h, tbuf=tbuf):
                for j in range(n_slices):
                    sl = pl.ds(j * _LANES, _LANES)
                    plsc.addupdate(tbuf.at[r, sl],
                                   pos_v[h * _CHUNK + r, sl])
            dst = out_hbm.at[pl.ds(b * S + sbase + h * _CHUNK, _CHUNK)]
            sd[c] = pltpu.async_copy(tbuf, dst, ssems[c % _NBUF])
        for c in range(n_chunks - _NBUF, n_chunks):
            sd[c].wait()

    return emb


def kernel(x, tok_table, pos_table):
    B, S = x.shape
    V, D = tok_table.shape
    out_flat = _build(B, S, D, V)(x.reshape(B * S), tok_table, pos_table)
    return out_flat.reshape(B, S, D)
